# baseline (speedup 1.0000x reference)
import functools
import math

import numpy as np

import jax
import jax.numpy as jnp
from jax import lax
from jax.experimental import pallas as pl
from jax.experimental.pallas import tpu as pltpu, tpu_sc as plsc

_FDIM = 64
_GRID = 513
_LANES = 16
_NC = 2
_NS = 16
_NW = _NC * _NS
_CHUNK = 128


def _sc_body(per_w, ntot, tail, xg_hbm, yg_hbm, tab_hbm, out_hbm,
             xbuf, ybuf, idxbuf, wbuf, rows, outbuf,
             gsem0, gsem1, osem0, osem1, xsem0, xsem1):
    cid = lax.axis_index("c")
    sid = lax.axis_index("s")
    wid = sid * _NC + cid
    first_x = wid * (ntot * _CHUNK)
    first_o = wid * per_w
    gsem = (gsem0, gsem1)
    osem = (osem0, osem1)
    xsem = (xsem0, xsem1)

    def fire_x(p, g):
        xbase = first_x + jnp.minimum(g, ntot - 1) * _CHUNK
        pltpu.async_copy(xg_hbm.at[pl.ds(xbase, _CHUNK)], xbuf.at[p], xsem[p])
        pltpu.async_copy(yg_hbm.at[pl.ds(xbase, _CHUNK)], ybuf.at[p], xsem[p])

    def wait_x(p):
        pltpu.make_async_copy(xg_hbm.at[pl.ds(first_x, _CHUNK)],
                              xbuf.at[p], xsem[p]).wait()
        pltpu.make_async_copy(yg_hbm.at[pl.ds(first_x, _CHUNK)],
                              ybuf.at[p], xsem[p]).wait()

    def stage(p, g):
        wait_x(p)
        for j in range(_CHUNK // _LANES):
            sl = pl.ds(_LANES * j, _LANES)
            gx = xbuf[p, sl]
            gy = ybuf[p, sl]
            ix = ((gx + 1.0) * float(_GRID) - 1.0) * 0.5
            iy = ((gy + 1.0) * float(_GRID) - 1.0) * 0.5
            x0 = (ix + 1.0).astype(jnp.int32) - 1
            y0 = (iy + 1.0).astype(jnp.int32) - 1
            wx1 = ix - x0.astype(jnp.float32)
            wx0 = 1.0 - wx1
            wy1 = iy - y0.astype(jnp.float32)
            wy0 = 1.0 - wy1
            wx0 = jnp.where(x0 >= 0, wx0, 0.0)
            wx1 = jnp.where(x0 <= _GRID - 2, wx1, 0.0)
            wy0 = jnp.where(y0 >= 0, wy0, 0.0)
            wy1 = jnp.where(y0 <= _GRID - 2, wy1, 0.0)
            xc0 = jnp.maximum(x0, 0)
            xc1 = jnp.minimum(x0 + 1, _GRID - 1)
            r0 = jnp.maximum(y0, 0) * _GRID
            r1 = jnp.minimum(y0 + 1, _GRID - 1) * _GRID
            idxbuf[p, 0, sl] = r0 + xc0
            idxbuf[p, 1, sl] = r0 + xc1
            idxbuf[p, 2, sl] = r1 + xc0
            idxbuf[p, 3, sl] = r1 + xc1
            wbuf[p, 0, sl] = wx0 * wy0
            wbuf[p, 1, sl] = wx1 * wy0
            wbuf[p, 2, sl] = wx0 * wy1
            wbuf[p, 3, sl] = wx1 * wy1
        fire_x(p, g + 2)
        for cc in range(4):
            pltpu.async_copy(tab_hbm.at[idxbuf.at[p, cc]], rows.at[p, cc],
                             gsem[p])

    def finish(p, g, drain, nrows=_CHUNK):
        obase = first_o + g * _CHUNK
        for cc in range(4):
            pltpu.make_async_copy(tab_hbm.at[idxbuf.at[p, cc]],
                                  rows.at[p, cc], gsem[p]).wait()

        if drain:
            pltpu.make_async_copy(outbuf.at[p],
                                  out_hbm.at[pl.ds(first_o, _CHUNK)],
                                  osem[p]).wait()

        himask = jnp.int32(-65536)

        def blend(g2, _):
            i0 = _LANES * g2
            wv = [wbuf[p, cc, pl.ds(i0, _LANES)] for cc in range(4)]
            for l in range(_LANES):
                i = i0 + l
                w = [wv[0][l], wv[1][l], wv[2][l], wv[3][l]]
                for h in range(_FDIM // (2 * _LANES)):
                    rw = [rows[p, cc, i, pl.ds(_LANES * h, _LANES)]
                          for cc in range(4)]
                    lo = [plsc.bitcast(v << 16, jnp.float32) for v in rw]
                    hi = [plsc.bitcast(v & himask, jnp.float32) for v in rw]
                    acc_a = (lo[0] * w[0] + lo[1] * w[1]
                             + lo[2] * w[2] + lo[3] * w[3])
                    acc_b = (hi[0] * w[0] + hi[1] * w[1]
                             + hi[2] * w[2] + hi[3] * w[3])
                    outbuf[p, i, pl.ds(2 * _LANES * h, _LANES)] = acc_a
                    outbuf[p, i, pl.ds(2 * _LANES * h + _LANES, _LANES)] = acc_b
            return 0

        lax.fori_loop(0, _CHUNK // _LANES, blend, 0)
        pltpu.async_copy(outbuf.at[p, pl.ds(0, nrows)],
                         out_hbm.at[pl.ds(obase, nrows)], osem[p])

    fire_x(0, 0)
    fire_x(1, 1)
    stage(0, 0)
    stage(1, 1)
    finish(0, 0, drain=False)
    stage(0, 2)
    finish(1, 1, drain=False)

    def loop_body(g2, _):
        ge = 2 * g2
        stage(1, ge + 1)
        finish(0, ge, drain=True)
        stage(0, ge + 2)
        finish(1, ge + 1, drain=True)
        return 0

    if ntot % 2:
        lax.fori_loop(1, (ntot - 1) // 2, loop_body, 0)
        finish(0, ntot - 1, drain=True, nrows=tail)
        last0, last1 = tail, _CHUNK
    else:
        lax.fori_loop(1, (ntot - 2) // 2, loop_body, 0)
        stage(1, ntot - 1)
        finish(0, ntot - 2, drain=True)
        finish(1, ntot - 1, drain=True, nrows=tail)
        last0, last1 = _CHUNK, tail

    pltpu.make_async_copy(outbuf.at[0, pl.ds(0, last0)],
                          out_hbm.at[pl.ds(first_o, last0)], osem0).wait()
    pltpu.make_async_copy(outbuf.at[1, pl.ds(0, last1)],
                          out_hbm.at[pl.ds(first_o, last1)], osem1).wait()
    wait_x(0)
    wait_x(1)


@functools.partial(jax.jit, static_argnames=("n", "per_w", "ntot", "tail"))
def _sc_sample(xg, yg, table, n, per_w, ntot, tail):
    mesh = plsc.VectorSubcoreMesh(core_axis_name="c", subcore_axis_name="s",
                                  num_cores=_NC, num_subcores=_NS)
    return pl.kernel(
        functools.partial(_sc_body, per_w, ntot, tail),
        out_type=jax.ShapeDtypeStruct((n, _FDIM), jnp.float32),
        mesh=mesh,
        compiler_params=pltpu.CompilerParams(use_tc_tiling_on_sc=False,
                                             needs_layout_passes=False),
        scratch_types=[
            pltpu.VMEM((2, _CHUNK), jnp.float32),
            pltpu.VMEM((2, _CHUNK), jnp.float32),
            pltpu.VMEM((2, 4, _CHUNK), jnp.int32),
            pltpu.VMEM((2, 4, _CHUNK), jnp.float32),
            pltpu.VMEM((2, 4, _CHUNK, _FDIM // 2), jnp.int32),
            pltpu.VMEM((2, _CHUNK, _FDIM), jnp.float32),
            pltpu.SemaphoreType.DMA,
            pltpu.SemaphoreType.DMA,
            pltpu.SemaphoreType.DMA,
            pltpu.SemaphoreType.DMA,
            pltpu.SemaphoreType.DMA,
            pltpu.SemaphoreType.DMA,
        ],
    )(xg, yg, table)


def kernel(x, fm):
    n = x.shape[0]
    assert n % _NW == 0, "point count must split evenly across the 32 tiles"
    per_w = n // _NW
    ntot = math.ceil(per_w / _CHUNK)
    assert ntot >= 3, "pipeline needs at least 3 chunks per tile"
    tail = per_w - (ntot - 1) * _CHUNK
    per_w_pad = ntot * _CHUNK
    perm = np.empty(_FDIM, np.int32)
    half = _FDIM // 4
    for h in range(2):
        for i in range(half):
            perm[2 * half * h + 2 * i] = 2 * half * h + i
            perm[2 * half * h + 2 * i + 1] = 2 * half * h + half + i
    table = fm[0].reshape(_FDIM, _GRID * _GRID).T
    table = table[:, perm].astype(jnp.bfloat16)
    table = jax.lax.bitcast_convert_type(
        table.reshape(_GRID * _GRID, _FDIM // 2, 2), jnp.int32)
    xr = x.reshape(_NW, per_w, 2)
    xp = jnp.pad(xr, ((0, 0), (0, per_w_pad - per_w), (0, 0)))
    xp = xp.reshape(_NW * per_w_pad, 2)
    return _sc_sample(xp[:, 0], xp[:, 1], table, n, per_w, ntot, tail)


# speedup vs baseline: 1.5282x; 1.5282x over previous
import functools
import math

import jax
import jax.numpy as jnp
from jax import lax
from jax.experimental import pallas as pl
from jax.experimental.pallas import tpu as pltpu, tpu_sc as plsc

_FDIM = 64
_GRID = 513
_NSITES = _GRID * _GRID
_RA = (_NSITES + 2) // 2
_LANES = 16
_NC = 2
_NS = 16
_NW = _NC * _NS
_CHUNK = 128


def _sc_body(per_w, ntot, xg_hbm, yg_hbm, tab_hbm, out_hbm,
             xbuf, ybuf, idxbuf, wbuf, rows, outbuf,
             gsem0, gsem1, osem0, osem1, xsem0, xsem1):
    cid = lax.axis_index("c")
    sid = lax.axis_index("s")
    wid = sid * _NC + cid
    astart = wid * per_w
    astart = pl.multiple_of(astart - (astart & 7), 8)
    aend = (wid + 1) * per_w
    aend = pl.multiple_of(aend - (aend & 7), 8)
    size_w = pl.multiple_of(aend - astart, 8)
    gsem = (gsem0, gsem1)
    osem = (osem0, osem1)
    xsem = (xsem0, xsem1)

    def chunk_base(g):
        return pl.multiple_of(
            astart + jnp.minimum(g * _CHUNK, size_w - _CHUNK), 8)

    def fire_x(p, g):
        base = chunk_base(jnp.minimum(g, ntot - 1))
        pltpu.async_copy(xg_hbm.at[pl.ds(base, _CHUNK)], xbuf.at[p], xsem[p])
        pltpu.async_copy(yg_hbm.at[pl.ds(base, _CHUNK)], ybuf.at[p], xsem[p])

    def wait_x(p):
        a8 = pl.multiple_of(astart, 8)
        pltpu.make_async_copy(xg_hbm.at[pl.ds(a8, _CHUNK)],
                              xbuf.at[p], xsem[p]).wait()
        pltpu.make_async_copy(yg_hbm.at[pl.ds(a8, _CHUNK)],
                              ybuf.at[p], xsem[p]).wait()

    def stage(p, g):
        wait_x(p)
        for j in range(_CHUNK // _LANES):
            sl = pl.ds(_LANES * j, _LANES)
            gx = xbuf[p, sl]
            gy = ybuf[p, sl]
            ix = ((gx + 1.0) * float(_GRID) - 1.0) * 0.5
            iy = ((gy + 1.0) * float(_GRID) - 1.0) * 0.5
            x0 = (ix + 1.0).astype(jnp.int32) - 1
            y0 = (iy + 1.0).astype(jnp.int32) - 1
            wx1 = ix - x0.astype(jnp.float32)
            wx0 = 1.0 - wx1
            wy1 = iy - y0.astype(jnp.float32)
            wy0 = 1.0 - wy1
            wx0 = jnp.where(x0 >= 0, wx0, 0.0)
            wx1 = jnp.where(x0 <= _GRID - 2, wx1, 0.0)
            wy0 = jnp.where(y0 >= 0, wy0, 0.0)
            wy1 = jnp.where(y0 <= _GRID - 2, wy1, 0.0)
            r0 = jnp.maximum(y0, 0) * _GRID
            r1 = jnp.minimum(y0 + 1, _GRID - 1) * _GRID
            s0 = r0 + x0
            s1 = r1 + x0
            i0 = ((s0 + 1) >> 1) + (1 - (s0 & 1)) * _RA
            i1 = ((s1 + 1) >> 1) + (1 - (s1 & 1)) * _RA
            idxbuf[p, 0, sl] = i0
            idxbuf[p, 1, sl] = i1
            wbuf[p, 0, sl] = wx0 * wy0
            wbuf[p, 1, sl] = wx1 * wy0
            wbuf[p, 2, sl] = wx0 * wy1
            wbuf[p, 3, sl] = wx1 * wy1
        fire_x(p, g + 2)
        for cc in range(2):
            pltpu.async_copy(tab_hbm.at[idxbuf.at[p, cc]], rows.at[p, cc],
                             gsem[p])

    def finish(p, g, drain):
        obase = chunk_base(g)
        for cc in range(2):
            pltpu.make_async_copy(tab_hbm.at[idxbuf.at[p, cc]],
                                  rows.at[p, cc], gsem[p]).wait()

        if drain:
            pltpu.make_async_copy(outbuf.at[p],
                                  out_hbm.at[pl.ds(astart, _CHUNK)],
                                  osem[p]).wait()

        def blend(g2, _):
            i0 = _LANES * g2
            wv = [wbuf[p, cc, pl.ds(i0, _LANES)] for cc in range(4)]
            for l in range(_LANES):
                i = i0 + l
                w0, w1, w2, w3 = wv[0][l], wv[1][l], wv[2][l], wv[3][l]
                for k in range(_FDIM // _LANES):
                    ks = pl.ds(_LANES * k, _LANES)
                    ks1 = pl.ds(_FDIM + _LANES * k, _LANES)
                    outbuf[p, i, ks] = (
                        rows[p, 0, i, ks] * w0 + rows[p, 0, i, ks1] * w1
                        + rows[p, 1, i, ks] * w2 + rows[p, 1, i, ks1] * w3)
            return 0

        lax.fori_loop(0, _CHUNK // _LANES, blend, 0)
        pltpu.async_copy(outbuf.at[p], out_hbm.at[pl.ds(obase, _CHUNK)],
                         osem[p])

    fire_x(0, 0)
    fire_x(1, 1)
    stage(0, 0)
    stage(1, 1)
    finish(0, 0, drain=False)
    stage(0, 2)
    finish(1, 1, drain=False)

    def loop_body(g2, _):
        ge = 2 * g2
        stage(1, ge + 1)
        finish(0, ge, drain=True)
        stage(0, ge + 2)
        finish(1, ge + 1, drain=True)
        return 0

    if ntot % 2:
        lax.fori_loop(1, (ntot - 1) // 2, loop_body, 0)
        finish(0, ntot - 1, drain=True)
    else:
        lax.fori_loop(1, (ntot - 2) // 2, loop_body, 0)
        stage(1, ntot - 1)
        finish(0, ntot - 2, drain=True)
        finish(1, ntot - 1, drain=True)

    pltpu.make_async_copy(outbuf.at[0],
                          out_hbm.at[pl.ds(astart, _CHUNK)], osem0).wait()
    pltpu.make_async_copy(outbuf.at[1],
                          out_hbm.at[pl.ds(astart, _CHUNK)], osem1).wait()
    wait_x(0)
    wait_x(1)


@functools.partial(jax.jit, static_argnames=("n", "per_w", "ntot"))
def _sc_sample(xg, yg, table, n, per_w, ntot):
    mesh = plsc.VectorSubcoreMesh(core_axis_name="c", subcore_axis_name="s",
                                  num_cores=_NC, num_subcores=_NS)
    return pl.kernel(
        functools.partial(_sc_body, per_w, ntot),
        out_type=jax.ShapeDtypeStruct((n, _FDIM), jnp.float32),
        mesh=mesh,
        compiler_params=pltpu.CompilerParams(use_tc_tiling_on_sc=True),
        scratch_types=[
            pltpu.VMEM((2, _CHUNK), jnp.float32),
            pltpu.VMEM((2, _CHUNK), jnp.float32),
            pltpu.VMEM((2, 2, _CHUNK), jnp.int32),
            pltpu.VMEM((2, 4, _CHUNK), jnp.float32),
            pltpu.VMEM((2, 2, _CHUNK, 2 * _FDIM), jnp.float32),
            pltpu.VMEM((2, _CHUNK, _FDIM), jnp.float32),
            pltpu.SemaphoreType.DMA,
            pltpu.SemaphoreType.DMA,
            pltpu.SemaphoreType.DMA,
            pltpu.SemaphoreType.DMA,
            pltpu.SemaphoreType.DMA,
            pltpu.SemaphoreType.DMA,
        ],
    )(xg, yg, table)


def kernel(x, fm):
    n = x.shape[0]
    assert n % _NW == 0 and n % 8 == 0, "fixed problem shape expected"
    per_w = n // _NW
    ntot = math.ceil(per_w / _CHUNK)
    assert ntot >= 4 and per_w >= _CHUNK + 8
    flat = fm[0].reshape(_FDIM, _NSITES).T.reshape(-1)
    zeros64 = jnp.zeros((_FDIM,), jnp.float32)
    flat1 = jnp.concatenate([zeros64, flat, zeros64])
    tab_a = flat1[:_RA * 2 * _FDIM].reshape(_RA, 2 * _FDIM)
    tab_b = flat1[_FDIM:_FDIM + _RA * 2 * _FDIM].reshape(_RA, 2 * _FDIM)
    table = jnp.concatenate([tab_a, tab_b])
    return _sc_sample(x[:, 0], x[:, 1], table, n, per_w, ntot)
